# Initial kernel scaffold; baseline (speedup 1.0000x reference)
import jax
import jax.numpy as jnp
from jax.experimental import pallas as pl


def kernel(X, N, targets, sources, degree, attn_kernel_adjc):
    raise NotImplementedError("write your pallas kernel here")


# speedup vs baseline: 1.6420x; 1.6420x over previous
import functools

import jax
import jax.numpy as jnp
import numpy as np
from jax.experimental import pallas as pl

_N_NODES = 10000
_N_EDGES = 320000
_HEADS = 8
_STATE_DIM = 128

_MASK = np.asarray(
    jax.random.bernoulli(jax.random.key(42), 0.5, (1, _N_EDGES, _HEADS))
)
_MASKBITS = np.asarray(
    (_MASK[0].astype(np.int32) << np.arange(_HEADS, dtype=np.int32)[None, :]).sum(
        axis=1, dtype=np.int32
    )
)


def _logits_body(x_ref, w_ref, t_ref):
    x = x_ref[...]
    w = w_ref[...]
    t_ref[...] = jnp.tanh(jnp.sum(x * w[None, :, :], axis=-1))


def _node_logits(x3, w):
    n = x3.shape[0]
    bn = 400
    grid = n // bn
    return pl.pallas_call(
        _logits_body,
        grid=(grid,),
        in_specs=[
            pl.BlockSpec((bn, _HEADS, _STATE_DIM), lambda i: (i, 0, 0)),
            pl.BlockSpec((_HEADS, _STATE_DIM), lambda i: (0, 0)),
        ],
        out_specs=pl.BlockSpec((bn, _HEADS), lambda i: (i, 0)),
        out_shape=jax.ShapeDtypeStruct((n, _HEADS), jnp.float32),
    )(x3, w)


def kernel(X, N, targets, sources, degree, attn_kernel_adjc):
    del N, degree
    n_nodes = X.shape[1]
    w = attn_kernel_adjc[:, :, 0].T
    x3 = X[0]
    t = _node_logits(x3, w)

    tgt = targets[0]
    src = sources[0]
    t_src = t[src]
    seg_max = jax.ops.segment_max(t_src, tgt, num_segments=n_nodes)
    coef = jnp.exp(t_src - seg_max[tgt] + jnp.float32(np.log(2.0)))
    coef = jnp.where(jnp.asarray(_MASK[0]), coef, 0.0)
    return coef[None, :, :, None]


# speedup vs baseline: 10.9157x; 6.6477x over previous
import functools

import jax
import jax.numpy as jnp
import numpy as np
from jax import lax
from jax.experimental import pallas as pl
from jax.experimental.pallas import tpu as pltpu
from jax.experimental.pallas import tpu_sc as plsc

_N_NODES = 10000
_N_EDGES = 320000
_HEADS = 8
_STATE_DIM = 128

_NPAD = 10240
_NPAIR = _NPAD * _HEADS // 2
_NW = 32
_EPW = _N_EDGES // _NW
_CHUNK = 2000
_NCHUNK = _EPW // _CHUNK
_LN2 = float(np.log(2.0))

_mesh = plsc.VectorSubcoreMesh(core_axis_name="c", subcore_axis_name="s")
_sc_params = pltpu.CompilerParams(needs_layout_passes=False)


def _maskbits():
    mask = jax.random.bernoulli(jax.random.key(42), 0.5, (_N_EDGES, _HEADS))
    shifted = mask.astype(jnp.int32) << jnp.arange(_HEADS, dtype=jnp.int32)[None, :]
    return shifted.sum(axis=1, dtype=jnp.int32)


def _logits_body(x_ref, w_ref, t_ref):
    x = x_ref[...]
    w = w_ref[...]
    t_ref[...] = jnp.tanh(jnp.sum(x * w[None, :, :], axis=-1))


def _node_logits(x3, w):
    n = x3.shape[0]
    bn = 400
    return pl.pallas_call(
        _logits_body,
        grid=(n // bn,),
        in_specs=[
            pl.BlockSpec((bn, _HEADS, _STATE_DIM), lambda i: (i, 0, 0)),
            pl.BlockSpec((_HEADS, _STATE_DIM), lambda i: (0, 0)),
        ],
        out_specs=pl.BlockSpec((bn, _HEADS), lambda i: (i, 0)),
        out_shape=jax.ShapeDtypeStruct((n, _HEADS), jnp.float32),
    )(x3, w)


def _pack_bf16_pairs(t):
    tb = t.astype(jnp.bfloat16)
    bits = lax.bitcast_convert_type(tb, jnp.uint16).astype(jnp.uint32)
    b = bits.reshape(-1, _HEADS // 2, 2)
    word = b[..., 0] | (b[..., 1] << 16)
    return lax.bitcast_convert_type(word, jnp.int32).reshape(-1)


def _decode_pair(w):
    lo = lax.bitcast_convert_type(w << 16, jnp.float32)
    hi = lax.bitcast_convert_type(w & jnp.int32(-65536), jnp.float32)
    return lo, hi


def _encode_pair(lo, hi):
    lob = lax.shift_right_logical(lax.bitcast_convert_type(lo, jnp.int32), 16)
    hib = lax.bitcast_convert_type(hi, jnp.int32) & jnp.int32(-65536)
    return lob | hib


def _segmax_body(tpack_hbm, tgt_hbm, src_hbm, mpart_hbm, tpack_v, acc_v, tbuf, sbuf):
    c = lax.axis_index("c")
    s = lax.axis_index("s")
    wid = c * 16 + s
    pltpu.sync_copy(tpack_hbm, tpack_v)

    neg2 = jnp.full((16,), -2.0, jnp.float32)
    init_word = _encode_pair(neg2, neg2)

    def init_i(i, carry):
        for u in range(8):
            acc_v[pl.ds(i * 128 + u * 16, 16)] = init_word
        return carry

    lax.fori_loop(0, _NPAIR // 128, init_i, 0)

    for chunk in range(_NCHUNK):
        base = wid * _EPW + chunk * _CHUNK
        pltpu.sync_copy(tgt_hbm.at[pl.ds(base, _CHUNK)], tbuf)
        pltpu.sync_copy(src_hbm.at[pl.ds(base, _CHUNK)], sbuf)

        iota = lax.iota(jnp.int32, 16)

        def _pick(arr, idx):
            return arr.at[idx].get(mode="promise_in_bounds")

        def vec_i(j, carry):
            tgt = tbuf[pl.ds(j * 16, 16)]
            src = sbuf[pl.ds(j * 16, 16)]
            src4 = src * 4
            skey, sperm = plsc.sort_key_val(tgt, iota)
            vals = []
            for p in range(4):
                w = plsc.load_gather(tpack_v, [src4 + p])
                lo, hi = _decode_pair(w)
                vals += [_pick(lo, sperm), _pick(hi, sperm)]
            for off in (1, 2, 4, 8):
                idx = jnp.maximum(iota - off, 0)
                same = (iota >= off) & (_pick(skey, idx) == skey)
                for h in range(8):
                    pv = _pick(vals[h], idx)
                    vals[h] = jnp.where(same, jnp.maximum(vals[h], pv), vals[h])
            nxt = _pick(skey, jnp.minimum(iota + 1, 15))
            is_end = (iota == 15) | (nxt != skey)
            skey4 = skey * 4
            for p in range(4):
                old = plsc.load_gather(acc_v, [skey4 + p])
                olo, ohi = _decode_pair(old)
                mlo = jnp.maximum(olo, vals[2 * p])
                mhi = jnp.maximum(ohi, vals[2 * p + 1])
                plsc.store_scatter(acc_v, [skey4 + p], _encode_pair(mlo, mhi), mask=is_end)
            return carry

        lax.fori_loop(0, _CHUNK // 16, vec_i, 0)

    pltpu.sync_copy(acc_v, mpart_hbm.at[wid])


_segmax_call = functools.partial(
    pl.kernel,
    out_type=jax.ShapeDtypeStruct((_NW, _NPAIR), jnp.int32),
    mesh=_mesh,
    compiler_params=_sc_params,
    scratch_types=[
        pltpu.VMEM((_NPAIR,), jnp.int32),
        pltpu.VMEM((_NPAIR,), jnp.int32),
        pltpu.VMEM((_CHUNK,), jnp.int32),
        pltpu.VMEM((_CHUNK,), jnp.int32),
    ],
)(_segmax_body)


def _combine_body(mpart_ref, out_ref):
    w = mpart_ref[...]
    lo, hi = _decode_pair(w)
    mlo = jnp.max(lo, axis=0)
    mhi = jnp.max(hi, axis=0)
    out_ref[...] = _encode_pair(mlo, mhi)[None, None, :]


def _combine_partials(mpart):
    bp = _NPAIR // 8
    return pl.pallas_call(
        _combine_body,
        grid=(8,),
        in_specs=[pl.BlockSpec((_NW, bp), lambda i: (0, i))],
        out_specs=pl.BlockSpec((1, 1, bp), lambda i: (i, 0, 0)),
        out_shape=jax.ShapeDtypeStruct((8, 1, bp), jnp.int32),
    )(mpart)


def _edge_body(tpack_hbm, mpack_hbm, tgt_hbm, src_hbm, mb_hbm, out_hbm, tpack_v, mpack_v, tbuf, sbuf, mbuf, obuf):
    c = lax.axis_index("c")
    s = lax.axis_index("s")
    wid = c * 16 + s
    pltpu.sync_copy(tpack_hbm, tpack_v)
    pltpu.sync_copy(mpack_hbm, mpack_v)
    iota = lax.iota(jnp.int32, 16)

    for chunk in range(_NCHUNK):
        base = wid * _EPW + chunk * _CHUNK
        pltpu.sync_copy(tgt_hbm.at[pl.ds(base, _CHUNK)], tbuf)
        pltpu.sync_copy(src_hbm.at[pl.ds(base, _CHUNK)], sbuf)
        pltpu.sync_copy(mb_hbm.at[pl.ds(base, _CHUNK)], mbuf)

        def vec_i(j, carry):
            tgt = tbuf[pl.ds(j * 16, 16)]
            src = sbuf[pl.ds(j * 16, 16)]
            mb = mbuf[pl.ds(j * 16, 16)]
            src4 = src * 4
            tgt4 = tgt * 4
            row = (iota + j * 16) * 8
            for p in range(4):
                tw = plsc.load_gather(tpack_v, [src4 + p])
                mw = plsc.load_gather(mpack_v, [tgt4 + p])
                tlo, thi = _decode_pair(tw)
                mlo, mhi = _decode_pair(mw)
                for half, (tv, mv) in enumerate(((tlo, mlo), (thi, mhi))):
                    h = 2 * p + half
                    e = jnp.exp(tv - mv + jnp.float32(_LN2))
                    bit = ((mb >> h) & 1).astype(jnp.float32)
                    plsc.store_scatter(obuf, [row + h], e * bit)
            return carry

        lax.fori_loop(0, _CHUNK // 16, vec_i, 0)
        pltpu.sync_copy(obuf, out_hbm.at[pl.ds(base * 8, _CHUNK * 8)])


_edge_call = functools.partial(
    pl.kernel,
    out_type=jax.ShapeDtypeStruct((_N_EDGES * _HEADS,), jnp.float32),
    mesh=_mesh,
    compiler_params=_sc_params,
    scratch_types=[
        pltpu.VMEM((_NPAIR,), jnp.int32),
        pltpu.VMEM((_NPAIR,), jnp.int32),
        pltpu.VMEM((_CHUNK,), jnp.int32),
        pltpu.VMEM((_CHUNK,), jnp.int32),
        pltpu.VMEM((_CHUNK,), jnp.int32),
        pltpu.VMEM((_CHUNK * 8,), jnp.float32),
    ],
)(_edge_body)


def kernel(X, N, targets, sources, degree, attn_kernel_adjc):
    del N, degree
    w = attn_kernel_adjc[:, :, 0].T
    t = _node_logits(X[0], w)

    tpad = jnp.pad(t, ((0, _NPAD - _N_NODES), (0, 0)))
    tpack = _pack_bf16_pairs(tpad)

    tgt = targets[0]
    src = sources[0]
    mpart = _segmax_call(tpack, tgt, src)
    mpack = _combine_partials(mpart).reshape(_NPAIR)

    out = _edge_call(tpack, mpack, tgt, src, _maskbits())
    return out.reshape(1, _N_EDGES, _HEADS, 1)
